# baseline (speedup 1.0000x reference)
import functools

import jax
import jax.numpy as jnp
from jax import lax
from jax.experimental import pallas as pl
from jax.experimental.pallas import tpu as pltpu
from jax.experimental.pallas import tpu_sc as plsc

N_NODES = 100000
D = 256
N_SEG = 256

TC_SEGS = 128
SC_SEGS = N_SEG - TC_SEGS

NUM_CORES = 2
NUM_SUBCORES = 16
NW = NUM_CORES * NUM_SUBCORES
SEG_PER_W = SC_SEGS // NW
LANES = 16
NVREG = D // LANES
CHUNK = 224

TC_CHUNK = 512
NEG_INF = float("-inf")


def _sc_body(feat_hbm, tbl_hbm, out_hbm, starts_vmem, buf0, buf1, acc,
             sem0, sem1):
  wid = lax.axis_index("s") * NUM_CORES + lax.axis_index("c")

  pltpu.sync_copy(tbl_hbm.at[wid], starts_vmem)
  sv = starts_vmem[...]

  neg_inf = jnp.full((LANES,), -jnp.inf, jnp.float32)
  for s in range(SEG_PER_W):
    for f in range(NVREG):
      acc[s, pl.ds(f * LANES, LANES)] = neg_inf

  row_start = sv[0]
  row_end = sv[SEG_PER_W]
  base = (row_start // 8) * 8
  n_chunks = (row_end - base + CHUNK - 1) // CHUNK

  def chunk_off(i):
    return pl.multiple_of(jnp.minimum(base + i * CHUNK, N_NODES - CHUNK), 8)

  bufs = (buf0, buf1)
  sems = (sem0, sem1)

  def start_load(i, slot):
    pltpu.async_copy(feat_hbm.at[pl.ds(chunk_off(i), CHUNK)], bufs[slot],
                     sems[slot])

  def wait_load(slot):
    pltpu.make_async_copy(
        feat_hbm.at[pl.ds(0, CHUNK)], bufs[slot], sems[slot]).wait()

  @pl.when(n_chunks > 0)
  def _prime():
    start_load(0, 0)

  def process(i, slot):
    o = chunk_off(i)
    buf = bufs[slot]
    for s in range(SEG_PER_W):
      lo = jnp.maximum(sv[s] - o, 0)
      hi = jnp.minimum(sv[s + 1] - o, CHUNK)
      hi = jnp.maximum(hi, lo)

      def row_body(r, a):
        return tuple(
            jnp.maximum(a[f], buf[r, pl.ds(f * LANES, LANES)])
            for f in range(NVREG)
        )

      a0 = tuple(acc[s, pl.ds(f * LANES, LANES)] for f in range(NVREG))
      a1 = plsc.parallel_loop(lo, hi, 1, unroll=4, carry=a0)(row_body)
      for f in range(NVREG):
        acc[s, pl.ds(f * LANES, LANES)] = a1[f]

  def outer(i2, _):
    for b in range(2):
      i = i2 * 2 + b

      @pl.when(i < n_chunks)
      def _():
        nxt = i + 1

        @pl.when(nxt < n_chunks)
        def _():
          start_load(nxt, (b + 1) % 2)

        wait_load(b)
        process(i, b)

    return 0

  lax.fori_loop(0, (n_chunks + 1) // 2, outer, 0)

  pltpu.sync_copy(acc, out_hbm.at[pl.ds(wid * SEG_PER_W, SEG_PER_W)])


def _tc_body(starts_ref, feat_hbm, out_ref, buf, sem):
  out_ref[...] = jnp.full((TC_SEGS, D), NEG_INF, jnp.float32)

  row_start = starts_ref[0]
  row_end = starts_ref[TC_SEGS]
  base = (row_start // 8) * 8
  n_chunks = (row_end - base + TC_CHUNK - 1) // TC_CHUNK

  def chunk_off(i):
    return pl.multiple_of(
        jnp.minimum(base + i * TC_CHUNK, N_NODES - TC_CHUNK), 8)

  def start_load(i, slot):
    pltpu.make_async_copy(
        feat_hbm.at[pl.ds(chunk_off(i), TC_CHUNK)], buf.at[slot],
        sem.at[slot]).start()

  def wait_load(slot):
    pltpu.make_async_copy(
        feat_hbm.at[pl.ds(0, TC_CHUNK)], buf.at[slot], sem.at[slot]).wait()

  @pl.when(n_chunks > 0)
  def _prime():
    start_load(0, 0)

  rows_iota = lax.broadcasted_iota(jnp.int32, (TC_CHUNK, 1), 0)

  def process(i, slot, s_ptr):
    o = chunk_off(i)
    e = o + TC_CHUNK
    rows = o + rows_iota

    def seg_body(s):
      rs = starts_ref[s]
      re = starts_ref[s + 1]
      m = (rows >= rs) & (rows < re)
      part = jnp.max(jnp.where(m, buf[slot], NEG_INF), axis=0)
      out_ref[pl.ds(s, 1), :] = jnp.maximum(out_ref[pl.ds(s, 1), :],
                                            part[None, :])

    def proc_cond(s):
      return (s < TC_SEGS) & (starts_ref[s] < e)

    def proc_step(s):
      seg_body(s)
      return s + 1

    lax.while_loop(proc_cond, proc_step, s_ptr)

    def adv_cond(s):
      return (s < TC_SEGS) & (starts_ref[s + 1] <= e)

    return lax.while_loop(adv_cond, lambda s: s + 1, s_ptr)

  def outer(i2, s_ptr):
    for b in range(2):
      i = i2 * 2 + b

      def do(s_ptr):
        nxt = i + 1

        @pl.when(nxt < n_chunks)
        def _():
          start_load(nxt, (b + 1) % 2)

        wait_load(b)
        return process(i, b, s_ptr)

      s_ptr = lax.cond(i < n_chunks, do, lambda s: s, s_ptr)
    return s_ptr

  lax.fori_loop(0, (n_chunks + 1) // 2, outer, 0)


@jax.jit
def kernel(feat, segment_ids):
  edges = jnp.arange(N_SEG + 1, dtype=jnp.int32)
  wnd = 500
  n_wnd = N_NODES // wnd
  windows = segment_ids.reshape(n_wnd, wnd)
  coarse = jnp.sum((windows[:, -1][None, :] < edges[:, None]).astype(jnp.int32),
                   axis=1)
  wclip = jnp.minimum(coarse, n_wnd - 1)
  brows = windows[wclip]
  inner = jnp.sum((brows < edges[:, None]).astype(jnp.int32), axis=1)
  starts = wclip * wnd + inner

  starts_pad = jnp.concatenate([starts, jnp.full((15,), N_NODES, jnp.int32)])

  wbase = TC_SEGS + jnp.arange(NW, dtype=jnp.int32) * SEG_PER_W
  tbl = starts_pad[wbase[:, None] + jnp.arange(16, dtype=jnp.int32)[None, :]]

  mesh = plsc.VectorSubcoreMesh(
      core_axis_name="c", subcore_axis_name="s",
      num_cores=NUM_CORES, num_subcores=NUM_SUBCORES)

  sc_fn = pl.kernel(
      _sc_body,
      out_type=jax.ShapeDtypeStruct((SC_SEGS, D), jnp.float32),
      mesh=mesh,
      scratch_types=[
          pltpu.VMEM((16,), jnp.int32),
          pltpu.VMEM((CHUNK, D), jnp.float32),
          pltpu.VMEM((CHUNK, D), jnp.float32),
          pltpu.VMEM((SEG_PER_W, D), jnp.float32),
          pltpu.SemaphoreType.DMA,
          pltpu.SemaphoreType.DMA,
      ],
  )
  sc_out = sc_fn(feat, tbl)

  tc_out = pl.pallas_call(
      _tc_body,
      out_shape=jax.ShapeDtypeStruct((TC_SEGS, D), jnp.float32),
      in_specs=[
          pl.BlockSpec(memory_space=pltpu.SMEM),
          pl.BlockSpec(memory_space=pl.ANY),
      ],
      out_specs=pl.BlockSpec(memory_space=pltpu.VMEM),
      scratch_shapes=[
          pltpu.VMEM((2, TC_CHUNK, D), jnp.float32),
          pltpu.SemaphoreType.DMA((2,)),
      ],
  )(starts_pad, feat)

  return jnp.concatenate([tc_out, sc_out], axis=0)


# speedup vs baseline: 1.4586x; 1.4586x over previous
import functools

import jax
import jax.numpy as jnp
from jax import lax
from jax.experimental import pallas as pl
from jax.experimental.pallas import tpu as pltpu
from jax.experimental.pallas import tpu_sc as plsc

N_NODES = 100000
D = 256
N_SEG = 256

TC_SEGS = 64
SC_SEGS = N_SEG - TC_SEGS

NUM_CORES = 2
NUM_SUBCORES = 16
NW = NUM_CORES * NUM_SUBCORES
SEG_PER_W = SC_SEGS // NW
LANES = 16
NVREG = D // LANES
CHUNK = 224

TC_CHUNK = 512
NEG_INF = float("-inf")


def _sc_body(feat_hbm, tbl_hbm, out_hbm, starts_vmem, buf0, buf1, acc,
             sem0, sem1):
  wid = lax.axis_index("s") * NUM_CORES + lax.axis_index("c")

  pltpu.sync_copy(tbl_hbm.at[wid], starts_vmem)
  sv = starts_vmem[...]

  neg_inf = jnp.full((LANES,), -jnp.inf, jnp.float32)
  for s in range(SEG_PER_W):
    for f in range(NVREG):
      acc[s, pl.ds(f * LANES, LANES)] = neg_inf

  row_start = sv[0]
  row_end = sv[SEG_PER_W]
  base = (row_start // 8) * 8
  n_chunks = (row_end - base + CHUNK - 1) // CHUNK

  def chunk_off(i):
    return pl.multiple_of(jnp.minimum(base + i * CHUNK, N_NODES - CHUNK), 8)

  bufs = (buf0, buf1)
  sems = (sem0, sem1)

  def start_load(i, slot):
    pltpu.async_copy(feat_hbm.at[pl.ds(chunk_off(i), CHUNK)], bufs[slot],
                     sems[slot])

  def wait_load(slot):
    pltpu.make_async_copy(
        feat_hbm.at[pl.ds(0, CHUNK)], bufs[slot], sems[slot]).wait()

  @pl.when(n_chunks > 0)
  def _prime():
    start_load(0, 0)

  def process(i, slot):
    o = chunk_off(i)
    buf = bufs[slot]
    for s in range(SEG_PER_W):
      lo = jnp.maximum(sv[s] - o, 0)
      hi = jnp.minimum(sv[s + 1] - o, CHUNK)
      hi = jnp.maximum(hi, lo)

      def row_body(r, a):
        return tuple(
            jnp.maximum(a[f], buf[r, pl.ds(f * LANES, LANES)])
            for f in range(NVREG)
        )

      a0 = tuple(acc[s, pl.ds(f * LANES, LANES)] for f in range(NVREG))
      a1 = plsc.parallel_loop(lo, hi, 1, unroll=4, carry=a0)(row_body)
      for f in range(NVREG):
        acc[s, pl.ds(f * LANES, LANES)] = a1[f]

  def outer(i2, _):
    for b in range(2):
      i = i2 * 2 + b

      @pl.when(i < n_chunks)
      def _():
        nxt = i + 1

        @pl.when(nxt < n_chunks)
        def _():
          start_load(nxt, (b + 1) % 2)

        wait_load(b)
        process(i, b)

    return 0

  lax.fori_loop(0, (n_chunks + 1) // 2, outer, 0)

  pltpu.sync_copy(acc, out_hbm.at[wid])


def _tc_body(starts_ref, feat_hbm, out_ref, buf, sem):
  out_ref[...] = jnp.full((TC_SEGS, D), NEG_INF, jnp.float32)

  row_start = starts_ref[0]
  row_end = starts_ref[TC_SEGS]
  base = (row_start // 8) * 8
  n_chunks = (row_end - base + TC_CHUNK - 1) // TC_CHUNK

  def chunk_off(i):
    return pl.multiple_of(
        jnp.minimum(base + i * TC_CHUNK, N_NODES - TC_CHUNK), 8)

  def start_load(i, slot):
    pltpu.make_async_copy(
        feat_hbm.at[pl.ds(chunk_off(i), TC_CHUNK)], buf.at[slot],
        sem.at[slot]).start()

  def wait_load(slot):
    pltpu.make_async_copy(
        feat_hbm.at[pl.ds(0, TC_CHUNK)], buf.at[slot], sem.at[slot]).wait()

  @pl.when(n_chunks > 0)
  def _prime():
    start_load(0, 0)

  rows_iota = lax.broadcasted_iota(jnp.int32, (TC_CHUNK, 1), 0)

  def process(i, slot, s_ptr):
    o = chunk_off(i)
    e = o + TC_CHUNK
    rows = o + rows_iota

    def seg_body(s):
      rs = starts_ref[s]
      re = starts_ref[s + 1]
      m = (rows >= rs) & (rows < re)
      part = jnp.max(jnp.where(m, buf[slot], NEG_INF), axis=0)
      out_ref[pl.ds(s, 1), :] = jnp.maximum(out_ref[pl.ds(s, 1), :],
                                            part[None, :])

    def proc_cond(s):
      return (s < TC_SEGS) & (starts_ref[s] < e)

    def proc_step(s):
      seg_body(s)
      return s + 1

    lax.while_loop(proc_cond, proc_step, s_ptr)

    def adv_cond(s):
      return (s < TC_SEGS) & (starts_ref[s + 1] <= e)

    return lax.while_loop(adv_cond, lambda s: s + 1, s_ptr)

  def outer(i2, s_ptr):
    for b in range(2):
      i = i2 * 2 + b

      def do(s_ptr):
        nxt = i + 1

        @pl.when(nxt < n_chunks)
        def _():
          start_load(nxt, (b + 1) % 2)

        wait_load(b)
        return process(i, b, s_ptr)

      s_ptr = lax.cond(i < n_chunks, do, lambda s: s, s_ptr)
    return s_ptr

  lax.fori_loop(0, (n_chunks + 1) // 2, outer, 0)


@jax.jit
def kernel(feat, segment_ids):
  edges = jnp.arange(N_SEG + 1, dtype=jnp.int32)
  wnd = 500
  n_wnd = N_NODES // wnd
  windows = segment_ids.reshape(n_wnd, wnd)
  coarse = jnp.sum((windows[:, -1][None, :] < edges[:, None]).astype(jnp.int32),
                   axis=1)
  wclip = jnp.minimum(coarse, n_wnd - 1)
  brows = windows[wclip]
  inner = jnp.sum((brows < edges[:, None]).astype(jnp.int32), axis=1)
  starts = wclip * wnd + inner

  starts_pad = jnp.concatenate([starts, jnp.full((15,), N_NODES, jnp.int32)])

  tbl = jnp.stack(
      [lax.dynamic_slice(starts_pad, (TC_SEGS + w * SEG_PER_W,), (16,))
       for w in range(NW)])

  mesh = plsc.VectorSubcoreMesh(
      core_axis_name="c", subcore_axis_name="s",
      num_cores=NUM_CORES, num_subcores=NUM_SUBCORES)

  sc_fn = pl.kernel(
      _sc_body,
      out_type=jax.ShapeDtypeStruct((NW, SEG_PER_W, D), jnp.float32),
      mesh=mesh,
      scratch_types=[
          pltpu.VMEM((16,), jnp.int32),
          pltpu.VMEM((CHUNK, D), jnp.float32),
          pltpu.VMEM((CHUNK, D), jnp.float32),
          pltpu.VMEM((SEG_PER_W, D), jnp.float32),
          pltpu.SemaphoreType.DMA,
          pltpu.SemaphoreType.DMA,
      ],
  )
  sc_out = sc_fn(feat, tbl).reshape(SC_SEGS, D)

  tc_out = pl.pallas_call(
      _tc_body,
      out_shape=jax.ShapeDtypeStruct((TC_SEGS, D), jnp.float32),
      in_specs=[
          pl.BlockSpec(memory_space=pltpu.SMEM),
          pl.BlockSpec(memory_space=pl.ANY),
      ],
      out_specs=pl.BlockSpec(memory_space=pltpu.VMEM),
      scratch_shapes=[
          pltpu.VMEM((2, TC_CHUNK, D), jnp.float32),
          pltpu.SemaphoreType.DMA((2,)),
      ],
  )(starts_pad, feat)

  return jnp.concatenate([tc_out, sc_out], axis=0)


# speedup vs baseline: 1.4672x; 1.0059x over previous
import functools

import jax
import jax.numpy as jnp
from jax import lax
from jax.experimental import pallas as pl
from jax.experimental.pallas import tpu as pltpu
from jax.experimental.pallas import tpu_sc as plsc

N_NODES = 100000
D = 256
N_SEG = 256

TC_SEGS = 64
SC_SEGS = N_SEG - TC_SEGS

NUM_CORES = 2
NUM_SUBCORES = 16
NW = NUM_CORES * NUM_SUBCORES
SEG_PER_W = SC_SEGS // NW
LANES = 16
NVREG = D // LANES
CHUNK = 224

TC_CHUNK = 512
NEG_INF = float("-inf")


def _sc_body(feat_hbm, tbl_hbm, out_hbm, starts_vmem, buf0, buf1, acc,
             sem0, sem1):
  wid = lax.axis_index("s") * NUM_CORES + lax.axis_index("c")

  pltpu.sync_copy(tbl_hbm.at[wid], starts_vmem)
  sv = starts_vmem[...]

  neg_inf = jnp.full((LANES,), -jnp.inf, jnp.float32)
  for s in range(SEG_PER_W):
    for f in range(NVREG):
      acc[s, pl.ds(f * LANES, LANES)] = neg_inf

  row_start = sv[0]
  row_end = sv[SEG_PER_W]
  base = (row_start // 8) * 8
  n_chunks = (row_end - base + CHUNK - 1) // CHUNK

  def chunk_off(i):
    return pl.multiple_of(jnp.minimum(base + i * CHUNK, N_NODES - CHUNK), 8)

  bufs = (buf0, buf1)
  sems = (sem0, sem1)

  def start_load(i, slot):
    pltpu.async_copy(feat_hbm.at[pl.ds(chunk_off(i), CHUNK)], bufs[slot],
                     sems[slot])

  def wait_load(slot):
    pltpu.make_async_copy(
        feat_hbm.at[pl.ds(0, CHUNK)], bufs[slot], sems[slot]).wait()

  @pl.when(n_chunks > 0)
  def _prime():
    start_load(0, 0)

  def process(i, slot):
    o = chunk_off(i)
    buf = bufs[slot]
    for s in range(SEG_PER_W):
      lo = jnp.maximum(sv[s] - o, 0)
      hi = jnp.minimum(sv[s + 1] - o, CHUNK)
      hi = jnp.maximum(hi, lo)

      def row_body(r, a):
        return tuple(
            jnp.maximum(a[f], buf[r, pl.ds(f * LANES, LANES)])
            for f in range(NVREG)
        )

      a0 = tuple(acc[s, pl.ds(f * LANES, LANES)] for f in range(NVREG))
      a1 = plsc.parallel_loop(lo, hi, 1, unroll=4, carry=a0)(row_body)
      for f in range(NVREG):
        acc[s, pl.ds(f * LANES, LANES)] = a1[f]

  def outer(i2, _):
    for b in range(2):
      i = i2 * 2 + b

      @pl.when(i < n_chunks)
      def _():
        nxt = i + 1

        @pl.when(nxt < n_chunks)
        def _():
          start_load(nxt, (b + 1) % 2)

        wait_load(b)
        process(i, b)

    return 0

  lax.fori_loop(0, (n_chunks + 1) // 2, outer, 0)

  pltpu.sync_copy(acc, out_hbm.at[wid])


def _tc_body(starts_ref, feat_hbm, out_ref, buf, gmax, sem):
  out_ref[...] = jnp.full((TC_SEGS, D), NEG_INF, jnp.float32)

  row_start = starts_ref[0]
  row_end = starts_ref[TC_SEGS]
  base = (row_start // 8) * 8
  n_chunks = (row_end - base + TC_CHUNK - 1) // TC_CHUNK

  def chunk_off(i):
    return pl.multiple_of(
        jnp.minimum(base + i * TC_CHUNK, N_NODES - TC_CHUNK), 8)

  def start_load(i, slot):
    pltpu.make_async_copy(
        feat_hbm.at[pl.ds(chunk_off(i), TC_CHUNK)], buf.at[slot],
        sem.at[slot]).start()

  def wait_load(slot):
    pltpu.make_async_copy(
        feat_hbm.at[pl.ds(0, TC_CHUNK)], buf.at[slot], sem.at[slot]).wait()

  @pl.when(n_chunks > 0)
  def _prime():
    start_load(0, 0)

  NGRP = TC_CHUNK // 8
  grp_iota = lax.broadcasted_iota(jnp.int32, (NGRP, 1), 0)
  row8_iota = lax.broadcasted_iota(jnp.int32, (8, 1), 0)

  def process(i, slot, s_ptr):
    o = chunk_off(i)
    e = o + TC_CHUNK

    for t in range(NGRP // 8):
      blk = buf[slot, pl.ds(t * 64, 64), :]
      gmax[pl.ds(t * 8, 8), :] = jnp.max(blk.reshape(8, 8, D), axis=1)

    def seg_body(s):
      rs = starts_ref[s]
      re = starts_ref[s + 1]
      lo = jnp.maximum(rs - o, 0)
      hi = jnp.minimum(re - o, TC_CHUNK)
      g_lo = (lo + 7) // 8
      g_hi = hi // 8
      gm = (grp_iota >= g_lo) & (grp_iota < g_hi)
      part = jnp.max(jnp.where(gm, gmax[...], NEG_INF), axis=0)
      for edge_grp, elo, ehi in (
          (lo // 8, lo, jnp.minimum(g_lo * 8, hi)),
          (jnp.minimum(g_hi, NGRP - 1), jnp.maximum(g_hi * 8, lo), hi),
      ):
        base8 = pl.multiple_of(edge_grp * 8, 8)
        rows8 = base8 + row8_iota
        em = (rows8 >= elo) & (rows8 < ehi)
        epart = jnp.max(
            jnp.where(em, buf[slot, pl.ds(base8, 8), :], NEG_INF), axis=0)
        part = jnp.maximum(part, epart)
      out_ref[pl.ds(s, 1), :] = jnp.maximum(out_ref[pl.ds(s, 1), :],
                                            part[None, :])

    def proc_cond(s):
      return (s < TC_SEGS) & (starts_ref[s] < e)

    def proc_step(s):
      seg_body(s)
      return s + 1

    lax.while_loop(proc_cond, proc_step, s_ptr)

    def adv_cond(s):
      return (s < TC_SEGS) & (starts_ref[s + 1] <= e)

    return lax.while_loop(adv_cond, lambda s: s + 1, s_ptr)

  def outer(i2, s_ptr):
    for b in range(2):
      i = i2 * 2 + b

      def do(s_ptr):
        nxt = i + 1

        @pl.when(nxt < n_chunks)
        def _():
          start_load(nxt, (b + 1) % 2)

        wait_load(b)
        return process(i, b, s_ptr)

      s_ptr = lax.cond(i < n_chunks, do, lambda s: s, s_ptr)
    return s_ptr

  lax.fori_loop(0, (n_chunks + 1) // 2, outer, 0)


@jax.jit
def kernel(feat, segment_ids):
  edges = jnp.arange(N_SEG + 1, dtype=jnp.int32)
  wnd = 500
  n_wnd = N_NODES // wnd
  windows = segment_ids.reshape(n_wnd, wnd)
  coarse = jnp.sum((windows[:, -1][None, :] < edges[:, None]).astype(jnp.int32),
                   axis=1)
  wclip = jnp.minimum(coarse, n_wnd - 1)
  brows = windows[wclip]
  inner = jnp.sum((brows < edges[:, None]).astype(jnp.int32), axis=1)
  starts = wclip * wnd + inner

  starts_pad = jnp.concatenate([starts, jnp.full((15,), N_NODES, jnp.int32)])

  tbl = jnp.stack(
      [lax.dynamic_slice(starts_pad, (TC_SEGS + w * SEG_PER_W,), (16,))
       for w in range(NW)])

  mesh = plsc.VectorSubcoreMesh(
      core_axis_name="c", subcore_axis_name="s",
      num_cores=NUM_CORES, num_subcores=NUM_SUBCORES)

  sc_fn = pl.kernel(
      _sc_body,
      out_type=jax.ShapeDtypeStruct((NW, SEG_PER_W, D), jnp.float32),
      mesh=mesh,
      scratch_types=[
          pltpu.VMEM((16,), jnp.int32),
          pltpu.VMEM((CHUNK, D), jnp.float32),
          pltpu.VMEM((CHUNK, D), jnp.float32),
          pltpu.VMEM((SEG_PER_W, D), jnp.float32),
          pltpu.SemaphoreType.DMA,
          pltpu.SemaphoreType.DMA,
      ],
  )
  sc_out = sc_fn(feat, tbl).reshape(SC_SEGS, D)

  tc_out = pl.pallas_call(
      _tc_body,
      out_shape=jax.ShapeDtypeStruct((TC_SEGS, D), jnp.float32),
      in_specs=[
          pl.BlockSpec(memory_space=pltpu.SMEM),
          pl.BlockSpec(memory_space=pl.ANY),
      ],
      out_specs=pl.BlockSpec(memory_space=pltpu.VMEM),
      scratch_shapes=[
          pltpu.VMEM((2, TC_CHUNK, D), jnp.float32),
          pltpu.VMEM((TC_CHUNK // 8, D), jnp.float32),
          pltpu.SemaphoreType.DMA((2,)),
      ],
  )(starts_pad, feat)

  return jnp.concatenate([tc_out, sc_out], axis=0)


# speedup vs baseline: 1.5575x; 1.0616x over previous
import functools

import jax
import jax.numpy as jnp
from jax import lax
from jax.experimental import pallas as pl
from jax.experimental.pallas import tpu as pltpu
from jax.experimental.pallas import tpu_sc as plsc

N_NODES = 100000
D = 256
N_SEG = 256

TC_SEGS = 64
SC_SEGS = N_SEG - TC_SEGS

NUM_CORES = 2
NUM_SUBCORES = 16
NW = NUM_CORES * NUM_SUBCORES
SEG_PER_W = SC_SEGS // NW
LANES = 16
NVREG = D // LANES
CHUNK = 224

TC_CHUNK = 512
TC_NBUF = 4
NEG_INF = float("-inf")


def _sc_body(feat_hbm, tbl_hbm, out_hbm, starts_vmem, buf0, buf1, acc,
             sem0, sem1):
  wid = lax.axis_index("s") * NUM_CORES + lax.axis_index("c")

  pltpu.sync_copy(tbl_hbm.at[wid], starts_vmem)
  sv = starts_vmem[...]

  neg_inf = jnp.full((LANES,), -jnp.inf, jnp.float32)
  for s in range(SEG_PER_W):
    for f in range(NVREG):
      acc[s, pl.ds(f * LANES, LANES)] = neg_inf

  row_start = sv[0]
  row_end = sv[SEG_PER_W]
  base = (row_start // 8) * 8
  n_chunks = (row_end - base + CHUNK - 1) // CHUNK

  def chunk_off(i):
    return pl.multiple_of(jnp.minimum(base + i * CHUNK, N_NODES - CHUNK), 8)

  bufs = (buf0, buf1)
  sems = (sem0, sem1)

  def start_load(i, slot):
    pltpu.async_copy(feat_hbm.at[pl.ds(chunk_off(i), CHUNK)], bufs[slot],
                     sems[slot])

  def wait_load(slot):
    pltpu.make_async_copy(
        feat_hbm.at[pl.ds(0, CHUNK)], bufs[slot], sems[slot]).wait()

  @pl.when(n_chunks > 0)
  def _prime():
    start_load(0, 0)

  def process(i, slot):
    o = chunk_off(i)
    buf = bufs[slot]
    for s in range(SEG_PER_W):
      lo = jnp.maximum(sv[s] - o, 0)
      hi = jnp.minimum(sv[s + 1] - o, CHUNK)
      hi = jnp.maximum(hi, lo)

      def row_body(r, a):
        return tuple(
            jnp.maximum(a[f], buf[r, pl.ds(f * LANES, LANES)])
            for f in range(NVREG)
        )

      a0 = tuple(acc[s, pl.ds(f * LANES, LANES)] for f in range(NVREG))
      a1 = plsc.parallel_loop(lo, hi, 1, unroll=4, carry=a0)(row_body)
      for f in range(NVREG):
        acc[s, pl.ds(f * LANES, LANES)] = a1[f]

  def outer(i2, _):
    for b in range(2):
      i = i2 * 2 + b

      @pl.when(i < n_chunks)
      def _():
        nxt = i + 1

        @pl.when(nxt < n_chunks)
        def _():
          start_load(nxt, (b + 1) % 2)

        wait_load(b)
        process(i, b)

    return 0

  lax.fori_loop(0, (n_chunks + 1) // 2, outer, 0)

  pltpu.sync_copy(acc, out_hbm.at[wid])


def _tc_body(starts_ref, feat_hbm, out_ref, buf, gmax, sem):
  out_ref[...] = jnp.full((TC_SEGS, D), NEG_INF, jnp.float32)

  row_start = starts_ref[0]
  row_end = starts_ref[TC_SEGS]
  base = (row_start // 8) * 8
  n_chunks = (row_end - base + TC_CHUNK - 1) // TC_CHUNK

  def chunk_off(i):
    return pl.multiple_of(
        jnp.minimum(base + i * TC_CHUNK, N_NODES - TC_CHUNK), 8)

  def start_load(i, slot):
    pltpu.make_async_copy(
        feat_hbm.at[pl.ds(chunk_off(i), TC_CHUNK)], buf.at[slot],
        sem.at[slot]).start()

  def wait_load(slot):
    pltpu.make_async_copy(
        feat_hbm.at[pl.ds(0, TC_CHUNK)], buf.at[slot], sem.at[slot]).wait()

  for j in range(TC_NBUF - 1):
    @pl.when(j < n_chunks)
    def _prime():
      start_load(j, j)

  NGRP = TC_CHUNK // 8
  grp_iota = lax.broadcasted_iota(jnp.int32, (NGRP, 1), 0)
  row8_iota = lax.broadcasted_iota(jnp.int32, (8, 1), 0)

  def process(i, slot, s_ptr):
    o = chunk_off(i)
    e = o + TC_CHUNK

    for t in range(NGRP // 8):
      blk = buf[slot, pl.ds(t * 64, 64), :]
      gmax[pl.ds(t * 8, 8), :] = jnp.max(blk.reshape(8, 8, D), axis=1)

    def seg_body(s):
      rs = starts_ref[s]
      re = starts_ref[s + 1]
      lo = jnp.maximum(rs - o, 0)
      hi = jnp.minimum(re - o, TC_CHUNK)
      g_lo = (lo + 7) // 8
      g_hi = hi // 8
      gm = (grp_iota >= g_lo) & (grp_iota < g_hi)
      part = jnp.max(jnp.where(gm, gmax[...], NEG_INF), axis=0)
      for edge_grp, elo, ehi in (
          (lo // 8, lo, jnp.minimum(g_lo * 8, hi)),
          (jnp.minimum(g_hi, NGRP - 1), jnp.maximum(g_hi * 8, lo), hi),
      ):
        base8 = pl.multiple_of(edge_grp * 8, 8)
        rows8 = base8 + row8_iota
        em = (rows8 >= elo) & (rows8 < ehi)
        epart = jnp.max(
            jnp.where(em, buf[slot, pl.ds(base8, 8), :], NEG_INF), axis=0)
        part = jnp.maximum(part, epart)
      out_ref[pl.ds(s, 1), :] = jnp.maximum(out_ref[pl.ds(s, 1), :],
                                            part[None, :])

    def proc_cond(s):
      return (s < TC_SEGS) & (starts_ref[s] < e)

    def proc_step(s):
      seg_body(s)
      return s + 1

    lax.while_loop(proc_cond, proc_step, s_ptr)

    def adv_cond(s):
      return (s < TC_SEGS) & (starts_ref[s + 1] <= e)

    return lax.while_loop(adv_cond, lambda s: s + 1, s_ptr)

  def outer(iN, s_ptr):
    for b in range(TC_NBUF):
      i = iN * TC_NBUF + b

      def do(s_ptr):
        nxt = i + TC_NBUF - 1

        @pl.when(nxt < n_chunks)
        def _():
          start_load(nxt, (b + TC_NBUF - 1) % TC_NBUF)

        wait_load(b)
        return process(i, b, s_ptr)

      s_ptr = lax.cond(i < n_chunks, do, lambda s: s, s_ptr)
    return s_ptr

  lax.fori_loop(0, (n_chunks + TC_NBUF - 1) // TC_NBUF, outer, 0)


@jax.jit
def kernel(feat, segment_ids):
  edges = jnp.arange(N_SEG + 1, dtype=jnp.int32)
  wnd = 500
  n_wnd = N_NODES // wnd
  windows = segment_ids.reshape(n_wnd, wnd)
  coarse = jnp.sum((windows[:, -1][None, :] < edges[:, None]).astype(jnp.int32),
                   axis=1)
  wclip = jnp.minimum(coarse, n_wnd - 1)
  brows = windows[wclip]
  inner = jnp.sum((brows < edges[:, None]).astype(jnp.int32), axis=1)
  starts = wclip * wnd + inner

  starts_pad = jnp.concatenate([starts, jnp.full((63,), N_NODES, jnp.int32)])

  a = starts_pad[TC_SEGS:]
  nblk = -(-16 // SEG_PER_W)
  tbl = jnp.concatenate(
      [a[SEG_PER_W * j:SEG_PER_W * (j + NW)].reshape(NW, SEG_PER_W)
       for j in range(nblk)], axis=1)[:, :16]

  mesh = plsc.VectorSubcoreMesh(
      core_axis_name="c", subcore_axis_name="s",
      num_cores=NUM_CORES, num_subcores=NUM_SUBCORES)

  sc_fn = pl.kernel(
      _sc_body,
      out_type=jax.ShapeDtypeStruct((NW, SEG_PER_W, D), jnp.float32),
      mesh=mesh,
      scratch_types=[
          pltpu.VMEM((16,), jnp.int32),
          pltpu.VMEM((CHUNK, D), jnp.float32),
          pltpu.VMEM((CHUNK, D), jnp.float32),
          pltpu.VMEM((SEG_PER_W, D), jnp.float32),
          pltpu.SemaphoreType.DMA,
          pltpu.SemaphoreType.DMA,
      ],
  )
  sc_out = sc_fn(feat, tbl).reshape(SC_SEGS, D)

  tc_out = pl.pallas_call(
      _tc_body,
      out_shape=jax.ShapeDtypeStruct((TC_SEGS, D), jnp.float32),
      in_specs=[
          pl.BlockSpec(memory_space=pltpu.SMEM),
          pl.BlockSpec(memory_space=pl.ANY),
      ],
      out_specs=pl.BlockSpec(memory_space=pltpu.VMEM),
      scratch_shapes=[
          pltpu.VMEM((TC_NBUF, TC_CHUNK, D), jnp.float32),
          pltpu.VMEM((TC_CHUNK // 8, D), jnp.float32),
          pltpu.SemaphoreType.DMA((TC_NBUF,)),
      ],
  )(starts_pad, feat)

  return jnp.concatenate([tc_out, sc_out], axis=0)


# speedup vs baseline: 1.5944x; 1.0237x over previous
import functools

import jax
import jax.numpy as jnp
from jax import lax
from jax.experimental import pallas as pl
from jax.experimental.pallas import tpu as pltpu
from jax.experimental.pallas import tpu_sc as plsc

N_NODES = 100000
D = 256
N_SEG = 256

TC_SEGS = 96
SC_SEGS = N_SEG - TC_SEGS

NUM_CORES = 2
NUM_SUBCORES = 16
NW = NUM_CORES * NUM_SUBCORES
SEG_PER_W = SC_SEGS // NW
LANES = 16
NVREG = D // LANES
CHUNK = 224

TC_CHUNK = 512
TC_NBUF = 4
NEG_INF = float("-inf")


def _sc_body(feat_hbm, tbl_hbm, out_hbm, starts_vmem, buf0, buf1, acc,
             sem0, sem1):
  wid = lax.axis_index("s") * NUM_CORES + lax.axis_index("c")

  pltpu.sync_copy(tbl_hbm.at[wid], starts_vmem)
  sv = starts_vmem[...]

  neg_inf = jnp.full((LANES,), -jnp.inf, jnp.float32)
  for s in range(SEG_PER_W):
    for f in range(NVREG):
      acc[s, pl.ds(f * LANES, LANES)] = neg_inf

  row_start = sv[0]
  row_end = sv[SEG_PER_W]
  base = (row_start // 8) * 8
  n_chunks = (row_end - base + CHUNK - 1) // CHUNK

  def chunk_off(i):
    return pl.multiple_of(jnp.minimum(base + i * CHUNK, N_NODES - CHUNK), 8)

  bufs = (buf0, buf1)
  sems = (sem0, sem1)

  def start_load(i, slot):
    pltpu.async_copy(feat_hbm.at[pl.ds(chunk_off(i), CHUNK)], bufs[slot],
                     sems[slot])

  def wait_load(slot):
    pltpu.make_async_copy(
        feat_hbm.at[pl.ds(0, CHUNK)], bufs[slot], sems[slot]).wait()

  @pl.when(n_chunks > 0)
  def _prime():
    start_load(0, 0)

  def process(i, slot):
    o = chunk_off(i)
    buf = bufs[slot]
    for s in range(SEG_PER_W):
      lo = jnp.maximum(sv[s] - o, 0)
      hi = jnp.minimum(sv[s + 1] - o, CHUNK)
      hi = jnp.maximum(hi, lo)

      def row_body(r, a):
        return tuple(
            jnp.maximum(a[f], buf[r, pl.ds(f * LANES, LANES)])
            for f in range(NVREG)
        )

      a0 = tuple(acc[s, pl.ds(f * LANES, LANES)] for f in range(NVREG))
      a1 = plsc.parallel_loop(lo, hi, 1, unroll=4, carry=a0)(row_body)
      for f in range(NVREG):
        acc[s, pl.ds(f * LANES, LANES)] = a1[f]

  def outer(i2, _):
    for b in range(2):
      i = i2 * 2 + b

      @pl.when(i < n_chunks)
      def _():
        nxt = i + 1

        @pl.when(nxt < n_chunks)
        def _():
          start_load(nxt, (b + 1) % 2)

        wait_load(b)
        process(i, b)

    return 0

  lax.fori_loop(0, (n_chunks + 1) // 2, outer, 0)

  pltpu.sync_copy(acc, out_hbm.at[wid])


def _tc_body(starts_ref, feat_hbm, out_ref, buf, gmax, sem):
  out_ref[...] = jnp.full((TC_SEGS, D), NEG_INF, jnp.float32)

  row_start = starts_ref[0]
  row_end = starts_ref[TC_SEGS]
  base = (row_start // 8) * 8
  n_chunks = (row_end - base + TC_CHUNK - 1) // TC_CHUNK

  def chunk_off(i):
    return pl.multiple_of(
        jnp.minimum(base + i * TC_CHUNK, N_NODES - TC_CHUNK), 8)

  def start_load(i, slot):
    pltpu.make_async_copy(
        feat_hbm.at[pl.ds(chunk_off(i), TC_CHUNK)], buf.at[slot],
        sem.at[slot]).start()

  def wait_load(slot):
    pltpu.make_async_copy(
        feat_hbm.at[pl.ds(0, TC_CHUNK)], buf.at[slot], sem.at[slot]).wait()

  for j in range(TC_NBUF - 1):
    @pl.when(j < n_chunks)
    def _prime():
      start_load(j, j)

  NGRP = TC_CHUNK // 8
  grp_iota = lax.broadcasted_iota(jnp.int32, (NGRP, 1), 0)
  row8_iota = lax.broadcasted_iota(jnp.int32, (8, 1), 0)

  def process(i, slot, s_ptr):
    o = chunk_off(i)
    e = o + TC_CHUNK

    for t in range(NGRP // 8):
      blk = buf[slot, pl.ds(t * 64, 64), :]
      gmax[pl.ds(t * 8, 8), :] = jnp.max(blk.reshape(8, 8, D), axis=1)

    def seg_body(s):
      rs = starts_ref[s]
      re = starts_ref[s + 1]
      lo = jnp.maximum(rs - o, 0)
      hi = jnp.minimum(re - o, TC_CHUNK)
      g_lo = (lo + 7) // 8
      g_hi = hi // 8
      gm = (grp_iota >= g_lo) & (grp_iota < g_hi)
      part = jnp.max(jnp.where(gm, gmax[...], NEG_INF), axis=0)
      for edge_grp, elo, ehi in (
          (lo // 8, lo, jnp.minimum(g_lo * 8, hi)),
          (jnp.minimum(g_hi, NGRP - 1), jnp.maximum(g_hi * 8, lo), hi),
      ):
        base8 = pl.multiple_of(edge_grp * 8, 8)
        rows8 = base8 + row8_iota
        em = (rows8 >= elo) & (rows8 < ehi)
        epart = jnp.max(
            jnp.where(em, buf[slot, pl.ds(base8, 8), :], NEG_INF), axis=0)
        part = jnp.maximum(part, epart)
      out_ref[pl.ds(s, 1), :] = jnp.maximum(out_ref[pl.ds(s, 1), :],
                                            part[None, :])

    def proc_cond(s):
      return (s < TC_SEGS) & (starts_ref[s] < e)

    def proc_step(s):
      seg_body(s)
      return s + 1

    lax.while_loop(proc_cond, proc_step, s_ptr)

    def adv_cond(s):
      return (s < TC_SEGS) & (starts_ref[s + 1] <= e)

    return lax.while_loop(adv_cond, lambda s: s + 1, s_ptr)

  def outer(iN, s_ptr):
    for b in range(TC_NBUF):
      i = iN * TC_NBUF + b

      def do(s_ptr):
        nxt = i + TC_NBUF - 1

        @pl.when(nxt < n_chunks)
        def _():
          start_load(nxt, (b + TC_NBUF - 1) % TC_NBUF)

        wait_load(b)
        return process(i, b, s_ptr)

      s_ptr = lax.cond(i < n_chunks, do, lambda s: s, s_ptr)
    return s_ptr

  lax.fori_loop(0, (n_chunks + TC_NBUF - 1) // TC_NBUF, outer, 0)


@jax.jit
def kernel(feat, segment_ids):
  edges = jnp.arange(N_SEG + 1, dtype=jnp.int32)
  wnd = 500
  n_wnd = N_NODES // wnd
  windows = segment_ids.reshape(n_wnd, wnd)
  coarse = jnp.sum((windows[:, -1][None, :] < edges[:, None]).astype(jnp.int32),
                   axis=1)
  wclip = jnp.minimum(coarse, n_wnd - 1)
  brows = windows[wclip]
  inner = jnp.sum((brows < edges[:, None]).astype(jnp.int32), axis=1)
  starts = wclip * wnd + inner

  starts_pad = jnp.concatenate([starts, jnp.full((63,), N_NODES, jnp.int32)])

  a = starts_pad[TC_SEGS:]
  nblk = -(-16 // SEG_PER_W)
  tbl = jnp.concatenate(
      [a[SEG_PER_W * j:SEG_PER_W * (j + NW)].reshape(NW, SEG_PER_W)
       for j in range(nblk)], axis=1)[:, :16]

  mesh = plsc.VectorSubcoreMesh(
      core_axis_name="c", subcore_axis_name="s",
      num_cores=NUM_CORES, num_subcores=NUM_SUBCORES)

  sc_fn = pl.kernel(
      _sc_body,
      out_type=jax.ShapeDtypeStruct((NW, SEG_PER_W, D), jnp.float32),
      mesh=mesh,
      scratch_types=[
          pltpu.VMEM((16,), jnp.int32),
          pltpu.VMEM((CHUNK, D), jnp.float32),
          pltpu.VMEM((CHUNK, D), jnp.float32),
          pltpu.VMEM((SEG_PER_W, D), jnp.float32),
          pltpu.SemaphoreType.DMA,
          pltpu.SemaphoreType.DMA,
      ],
  )
  sc_out = sc_fn(feat, tbl).reshape(SC_SEGS, D)

  tc_out = pl.pallas_call(
      _tc_body,
      out_shape=jax.ShapeDtypeStruct((TC_SEGS, D), jnp.float32),
      in_specs=[
          pl.BlockSpec(memory_space=pltpu.SMEM),
          pl.BlockSpec(memory_space=pl.ANY),
      ],
      out_specs=pl.BlockSpec(memory_space=pltpu.VMEM),
      scratch_shapes=[
          pltpu.VMEM((TC_NBUF, TC_CHUNK, D), jnp.float32),
          pltpu.VMEM((TC_CHUNK // 8, D), jnp.float32),
          pltpu.SemaphoreType.DMA((TC_NBUF,)),
      ],
  )(starts_pad, feat)

  return jnp.concatenate([tc_out, sc_out], axis=0)
